# baseline (speedup 1.0000x reference)
import functools

import jax
import jax.numpy as jnp
from jax import lax
from jax.experimental import pallas as pl
from jax.experimental.pallas import tpu as pltpu
from jax.experimental.pallas import tpu_sc as plsc

N_NODES = 10000
N_EDGES = 320000
D = 128
DH = D // 2

NUM_CORES = 2
NUM_SUBCORES = 16
E_PER_TILE = N_EDGES // NUM_SUBCORES
CHUNK = 40
N_CHUNKS = E_PER_TILE // CHUNK
NBUF = 4
LA = 2
ROWS_PER_TILE = 624
TAIL_ROWS = N_NODES - NUM_SUBCORES * ROWS_PER_TILE
DROWS = 104


def _sc_agg(src0, src1, edge, src_idx, dst_idx, z64, z16):
    mesh = plsc.VectorSubcoreMesh(core_axis_name="c", subcore_axis_name="s")

    scratch = (
        [pltpu.VMEM((1, CHUNK), jnp.int32)] * NBUF
        + [pltpu.VMEM((1, CHUNK), jnp.int32)] * NBUF
        + [pltpu.VMEM((CHUNK, DH), jnp.float32)] * NBUF
        + [pltpu.VMEM((CHUNK, DH), jnp.float32)] * NBUF
        + [
            pltpu.VMEM((CHUNK, 16), jnp.float32),
            pltpu.VMEM((DROWS, DH), jnp.float32),
            pltpu.VMEM((DROWS, 16), jnp.float32),
            pltpu.VMEM_SHARED((N_NODES, DH), jnp.float32),
            pltpu.VMEM_SHARED((N_NODES, 16), jnp.float32),
        ]
        + [pltpu.SemaphoreType.DMA] * (3 * NBUF)
    )

    @functools.partial(
        pl.kernel,
        mesh=mesh,
        out_type=jax.ShapeDtypeStruct((N_NODES, D), jnp.float32),
        scratch_types=scratch,
        compiler_params=pltpu.CompilerParams(use_tc_tiling_on_sc=False,
                                             needs_layout_passes=False),
    )
    def ker(src0_hbm, src1_hbm, edge_hbm, sidx_hbm, didx_hbm, z64_hbm,
            z16_hbm, out_hbm, *scr):
        sidx = scr[0:NBUF]
        didx = scr[NBUF:2 * NBUF]
        srows = scr[2 * NBUF:3 * NBUF]
        erows = scr[3 * NBUF:4 * NBUF]
        ones_v, sbuf, dbuf, ssum, sdeg = scr[4 * NBUF:4 * NBUF + 5]
        sem_i = scr[4 * NBUF + 5:5 * NBUF + 5]
        sem_g = scr[5 * NBUF + 5:6 * NBUF + 5]
        sem_s = scr[6 * NBUF + 5:7 * NBUF + 5]

        cid = lax.axis_index("c")
        sid = lax.axis_index("s")

        onevec = jnp.where(lax.iota(jnp.int32, 16) == 0, 1.0, 0.0)

        def ones_body(j, _):
            ones_v[j, pl.ds(0, 16)] = onevec
            return 0

        lax.fori_loop(0, CHUNK, ones_body, 0)

        row0 = sid * ROWS_PER_TILE
        pltpu.sync_copy(z64_hbm, ssum.at[pl.ds(row0, ROWS_PER_TILE)])
        pltpu.sync_copy(z16_hbm, sdeg.at[pl.ds(row0, ROWS_PER_TILE)])

        tail0 = NUM_SUBCORES * ROWS_PER_TILE

        @pl.when(sid == NUM_SUBCORES - 1)
        def _zero_tail():
            pltpu.sync_copy(z64_hbm.at[pl.ds(0, TAIL_ROWS)],
                            ssum.at[pl.ds(tail0, TAIL_ROWS)])
            pltpu.sync_copy(z16_hbm.at[pl.ds(0, TAIL_ROWS)],
                            sdeg.at[pl.ds(tail0, TAIL_ROWS)])

        plsc.subcore_barrier()

        def issue_idx(c, b):
            base = sid * E_PER_TILE + c * CHUNK
            pltpu.async_copy(sidx_hbm.at[pl.ds(base, CHUNK)], sidx[b].at[0],
                             sem_i[b])
            pltpu.async_copy(didx_hbm.at[pl.ds(base, CHUNK)], didx[b].at[0],
                             sem_i[b])

        def wait_idx(c, b):
            base = sid * E_PER_TILE + c * CHUNK
            pltpu.make_async_copy(sidx_hbm.at[pl.ds(base, CHUNK)],
                                  sidx[b].at[0], sem_i[b]).wait()
            pltpu.make_async_copy(didx_hbm.at[pl.ds(base, CHUNK)],
                                  didx[b].at[0], sem_i[b]).wait()

        def issue_loads(c, b):
            idx = sidx[b].at[0]
            row_sl = pl.ds(sid * E_PER_TILE + c * CHUNK, CHUNK)

            @pl.when(cid == 0)
            def _g0():
                pltpu.async_copy(src0_hbm.at[idx], srows[b], sem_g[b])
                pltpu.async_copy(edge_hbm.at[row_sl, pl.ds(0, DH)],
                                 erows[b], sem_g[b])

            @pl.when(cid == 1)
            def _g1():
                pltpu.async_copy(src1_hbm.at[idx], srows[b], sem_g[b])
                pltpu.async_copy(edge_hbm.at[row_sl, pl.ds(DH, DH)],
                                 erows[b], sem_g[b])

        def wait_loads(c, b):
            pltpu.make_async_copy(src0_hbm.at[sidx[b].at[0]], srows[b],
                                  sem_g[b]).wait()
            pltpu.make_async_copy(
                edge_hbm.at[pl.ds(sid * E_PER_TILE + c * CHUNK, CHUNK),
                            pl.ds(0, DH)],
                erows[b], sem_g[b]).wait()

        def issue_scatters(c, b):
            idx = didx[b].at[0]
            pltpu.async_copy(srows[b], ssum.at[idx], sem_s[b], add=True)
            pltpu.async_copy(erows[b], ssum.at[idx], sem_s[b], add=True)
            pltpu.async_copy(ones_v, sdeg.at[idx], sem_s[b], add=True)

        def wait_scatters(c, b):
            idx = didx[b].at[0]
            pltpu.make_async_copy(srows[b], ssum.at[idx], sem_s[b]).wait()
            pltpu.make_async_copy(erows[b], ssum.at[idx], sem_s[b]).wait()
            pltpu.make_async_copy(ones_v, sdeg.at[idx], sem_s[b]).wait()

        for c in range(LA):
            issue_idx(c, c % NBUF)
        for c in range(LA):
            wait_idx(c, c % NBUF)
            issue_loads(c, c % NBUF)
        issue_idx(LA, LA % NBUF)

        def ring_body(g, _):
            for b in range(NBUF):
                c = g * NBUF + b
                wait_loads(c, b)
                issue_scatters(c, b)

                c3 = c + LA + 1
                b3 = (b + LA + 1) % NBUF

                @pl.when(c3 < N_CHUNKS)
                def _idx_prefetch():
                    issue_idx(c3, b3)

                c2 = c + LA
                b2 = (b + LA) % NBUF

                @pl.when(c2 < N_CHUNKS)
                def _load_prefetch():
                    @pl.when(c >= NBUF - LA)
                    def _drain_prev():
                        wait_scatters(c - (NBUF - LA), b2)

                    wait_idx(c2, b2)
                    issue_loads(c2, b2)

            return 0

        lax.fori_loop(0, N_CHUNKS // NBUF, ring_body, 0)

        for k in range(NBUF):
            cw = N_CHUNKS - NBUF + k
            wait_scatters(cw, cw % NBUF)

        plsc.subcore_barrier()

        col_sl = pl.ds(0, DH)

        def divide_rows(r, n):
            pltpu.sync_copy(ssum.at[pl.ds(r, n)], sbuf.at[pl.ds(0, n)])
            pltpu.sync_copy(sdeg.at[pl.ds(r, n)], dbuf.at[pl.ds(0, n)])

            def row_body(j, _):
                dvec = plsc.cumsum(dbuf[j, pl.ds(0, 16)])
                rv = 1.0 / jnp.maximum(dvec, 1.0)
                for k in range(DH // 16):
                    sl = pl.ds(k * 16, 16)
                    sbuf[j, sl] = sbuf[j, sl] * rv
                return 0

            lax.fori_loop(0, n, row_body, 0)

            @pl.when(cid == 0)
            def _w0():
                pltpu.sync_copy(sbuf.at[pl.ds(0, n)],
                                out_hbm.at[pl.ds(r, n), pl.ds(0, DH)])

            @pl.when(cid == 1)
            def _w1():
                pltpu.sync_copy(sbuf.at[pl.ds(0, n)],
                                out_hbm.at[pl.ds(r, n), pl.ds(DH, DH)])

        for i in range(ROWS_PER_TILE // DROWS):
            divide_rows(row0 + i * DROWS, DROWS)

        @pl.when(sid == NUM_SUBCORES - 1)
        def _div_tail():
            divide_rows(tail0, TAIL_ROWS)

    return ker(src0, src1, edge, src_idx, dst_idx, z64, z16)


@jax.jit
def kernel(src_embedding, edge_embedding, edge_index):
    src_idx = edge_index[0].astype(jnp.int32)
    dst_idx = edge_index[1].astype(jnp.int32)
    src0 = src_embedding[:, :DH]
    src1 = src_embedding[:, DH:]
    z64 = jnp.zeros((ROWS_PER_TILE, DH), jnp.float32)
    z16 = jnp.zeros((ROWS_PER_TILE, 16), jnp.float32)
    return _sc_agg(src0, src1, edge_embedding, src_idx, dst_idx, z64, z16)


# speedup vs baseline: 1.1429x; 1.1429x over previous
import functools

import jax
import jax.numpy as jnp
from jax import lax
from jax.experimental import pallas as pl
from jax.experimental.pallas import tpu as pltpu
from jax.experimental.pallas import tpu_sc as plsc

N_NODES = 10000
N_EDGES = 320000
D = 128
DH = D // 2

NUM_CORES = 2
NUM_SUBCORES = 16
E_PER_TILE = N_EDGES // NUM_SUBCORES
CHUNK = 40
N_CHUNKS = E_PER_TILE // CHUNK
NBUF = 5
LA = 3
ROWS_PER_TILE = 624
TAIL_ROWS = N_NODES - NUM_SUBCORES * ROWS_PER_TILE
DROWS = 48


def _sc_agg(src0, src1, edge, src_idx, dst_idx, z64, z16):
    mesh = plsc.VectorSubcoreMesh(core_axis_name="c", subcore_axis_name="s")

    scratch = (
        [pltpu.VMEM((1, CHUNK), jnp.int32)] * NBUF
        + [pltpu.VMEM((1, CHUNK), jnp.int32)] * NBUF
        + [pltpu.VMEM((CHUNK, DH), jnp.float32)] * NBUF
        + [pltpu.VMEM((CHUNK, DH), jnp.float32)] * NBUF
        + [
            pltpu.VMEM((CHUNK, 16), jnp.float32),
            pltpu.VMEM((DROWS, DH), jnp.float32),
            pltpu.VMEM((DROWS, 16), jnp.float32),
            pltpu.VMEM_SHARED((N_NODES, DH), jnp.float32),
            pltpu.VMEM_SHARED((N_NODES, 16), jnp.float32),
        ]
        + [pltpu.SemaphoreType.DMA] * (3 * NBUF)
    )

    @functools.partial(
        pl.kernel,
        mesh=mesh,
        out_type=jax.ShapeDtypeStruct((N_NODES, D), jnp.float32),
        scratch_types=scratch,
        compiler_params=pltpu.CompilerParams(use_tc_tiling_on_sc=False,
                                             needs_layout_passes=False),
    )
    def ker(src0_hbm, src1_hbm, edge_hbm, sidx_hbm, didx_hbm, z64_hbm,
            z16_hbm, out_hbm, *scr):
        sidx = scr[0:NBUF]
        didx = scr[NBUF:2 * NBUF]
        srows = scr[2 * NBUF:3 * NBUF]
        erows = scr[3 * NBUF:4 * NBUF]
        ones_v, sbuf, dbuf, ssum, sdeg = scr[4 * NBUF:4 * NBUF + 5]
        sem_i = scr[4 * NBUF + 5:5 * NBUF + 5]
        sem_g = scr[5 * NBUF + 5:6 * NBUF + 5]
        sem_s = scr[6 * NBUF + 5:7 * NBUF + 5]

        cid = lax.axis_index("c")
        sid = lax.axis_index("s")

        onevec = jnp.where(lax.iota(jnp.int32, 16) == 0, 1.0, 0.0)

        def ones_body(j, _):
            ones_v[j, pl.ds(0, 16)] = onevec
            return 0

        lax.fori_loop(0, CHUNK, ones_body, 0)

        row0 = sid * ROWS_PER_TILE
        pltpu.sync_copy(z64_hbm, ssum.at[pl.ds(row0, ROWS_PER_TILE)])
        pltpu.sync_copy(z16_hbm, sdeg.at[pl.ds(row0, ROWS_PER_TILE)])

        tail0 = NUM_SUBCORES * ROWS_PER_TILE

        @pl.when(sid == NUM_SUBCORES - 1)
        def _zero_tail():
            pltpu.sync_copy(z64_hbm.at[pl.ds(0, TAIL_ROWS)],
                            ssum.at[pl.ds(tail0, TAIL_ROWS)])
            pltpu.sync_copy(z16_hbm.at[pl.ds(0, TAIL_ROWS)],
                            sdeg.at[pl.ds(tail0, TAIL_ROWS)])

        plsc.subcore_barrier()

        def issue_idx(c, b):
            base = sid * E_PER_TILE + c * CHUNK
            pltpu.async_copy(sidx_hbm.at[pl.ds(base, CHUNK)], sidx[b].at[0],
                             sem_i[b])
            pltpu.async_copy(didx_hbm.at[pl.ds(base, CHUNK)], didx[b].at[0],
                             sem_i[b])

        def wait_idx(c, b):
            base = sid * E_PER_TILE + c * CHUNK
            pltpu.make_async_copy(sidx_hbm.at[pl.ds(base, CHUNK)],
                                  sidx[b].at[0], sem_i[b]).wait()
            pltpu.make_async_copy(didx_hbm.at[pl.ds(base, CHUNK)],
                                  didx[b].at[0], sem_i[b]).wait()

        def issue_loads(c, b):
            idx = sidx[b].at[0]
            row_sl = pl.ds(sid * E_PER_TILE + c * CHUNK, CHUNK)

            @pl.when(cid == 0)
            def _g0():
                pltpu.async_copy(src0_hbm.at[idx], srows[b], sem_g[b])
                pltpu.async_copy(edge_hbm.at[row_sl, pl.ds(0, DH)],
                                 erows[b], sem_g[b])

            @pl.when(cid == 1)
            def _g1():
                pltpu.async_copy(src1_hbm.at[idx], srows[b], sem_g[b])
                pltpu.async_copy(edge_hbm.at[row_sl, pl.ds(DH, DH)],
                                 erows[b], sem_g[b])

        def wait_loads(c, b):
            pltpu.make_async_copy(src0_hbm.at[sidx[b].at[0]], srows[b],
                                  sem_g[b]).wait()
            pltpu.make_async_copy(
                edge_hbm.at[pl.ds(sid * E_PER_TILE + c * CHUNK, CHUNK),
                            pl.ds(0, DH)],
                erows[b], sem_g[b]).wait()

        def issue_scatters(c, b):
            idx = didx[b].at[0]
            pltpu.async_copy(srows[b], ssum.at[idx], sem_s[b], add=True)
            pltpu.async_copy(erows[b], ssum.at[idx], sem_s[b], add=True)
            pltpu.async_copy(ones_v, sdeg.at[idx], sem_s[b], add=True)

        def wait_scatters(c, b):
            idx = didx[b].at[0]
            pltpu.make_async_copy(srows[b], ssum.at[idx], sem_s[b]).wait()
            pltpu.make_async_copy(erows[b], ssum.at[idx], sem_s[b]).wait()
            pltpu.make_async_copy(ones_v, sdeg.at[idx], sem_s[b]).wait()

        for c in range(LA):
            issue_idx(c, c % NBUF)
        for c in range(LA):
            wait_idx(c, c % NBUF)
            issue_loads(c, c % NBUF)
        issue_idx(LA, LA % NBUF)

        def ring_body(g, _):
            for b in range(NBUF):
                c = g * NBUF + b
                wait_loads(c, b)
                issue_scatters(c, b)

                c3 = c + LA + 1
                b3 = (b + LA + 1) % NBUF

                @pl.when(c3 < N_CHUNKS)
                def _idx_prefetch():
                    issue_idx(c3, b3)

                c2 = c + LA
                b2 = (b + LA) % NBUF

                @pl.when(c2 < N_CHUNKS)
                def _load_prefetch():
                    @pl.when(c >= NBUF - LA)
                    def _drain_prev():
                        wait_scatters(c - (NBUF - LA), b2)

                    wait_idx(c2, b2)
                    issue_loads(c2, b2)

            return 0

        lax.fori_loop(0, N_CHUNKS // NBUF, ring_body, 0)

        for k in range(NBUF):
            cw = N_CHUNKS - NBUF + k
            wait_scatters(cw, cw % NBUF)

        plsc.subcore_barrier()

        col_sl = pl.ds(0, DH)

        def divide_rows(r, n):
            pltpu.sync_copy(ssum.at[pl.ds(r, n)], sbuf.at[pl.ds(0, n)])
            pltpu.sync_copy(sdeg.at[pl.ds(r, n)], dbuf.at[pl.ds(0, n)])

            def row_body(j, _):
                dvec = plsc.cumsum(dbuf[j, pl.ds(0, 16)])
                rv = 1.0 / jnp.maximum(dvec, 1.0)
                for k in range(DH // 16):
                    sl = pl.ds(k * 16, 16)
                    sbuf[j, sl] = sbuf[j, sl] * rv
                return 0

            lax.fori_loop(0, n, row_body, 0)

            @pl.when(cid == 0)
            def _w0():
                pltpu.sync_copy(sbuf.at[pl.ds(0, n)],
                                out_hbm.at[pl.ds(r, n), pl.ds(0, DH)])

            @pl.when(cid == 1)
            def _w1():
                pltpu.sync_copy(sbuf.at[pl.ds(0, n)],
                                out_hbm.at[pl.ds(r, n), pl.ds(DH, DH)])

        for i in range(ROWS_PER_TILE // DROWS):
            divide_rows(row0 + i * DROWS, DROWS)

        @pl.when(sid == NUM_SUBCORES - 1)
        def _div_tail():
            divide_rows(tail0, TAIL_ROWS)

    return ker(src0, src1, edge, src_idx, dst_idx, z64, z16)


@jax.jit
def kernel(src_embedding, edge_embedding, edge_index):
    src_idx = edge_index[0].astype(jnp.int32)
    dst_idx = edge_index[1].astype(jnp.int32)
    src0 = src_embedding[:, :DH]
    src1 = src_embedding[:, DH:]
    z64 = jnp.zeros((ROWS_PER_TILE, DH), jnp.float32)
    z16 = jnp.zeros((ROWS_PER_TILE, 16), jnp.float32)
    return _sc_agg(src0, src1, edge_embedding, src_idx, dst_idx, z64, z16)
